# baseline (speedup 1.0000x reference)
import functools

import jax
import jax.numpy as jnp
from jax import lax
from jax.experimental import pallas as pl
from jax.experimental.pallas import tpu as pltpu
from jax.experimental.pallas import tpu_sc as plsc


def _build_lookup(N, V, D, L, B):
    info = plsc.get_sparse_core_info()
    nc, ns = info.num_cores, info.num_subcores
    nw = nc * ns
    per_w = N // nw
    CHUNK = L
    n_chunks = per_w // CHUNK
    assert n_chunks * CHUNK == per_w and n_chunks % 2 == 0
    assert n_chunks * nw == B
    LP = ((L + 15) // 16) * 16
    splits = []
    off = 0
    while off < CHUNK:
        g = min(128, CHUNK - off)
        splits.append((off, g))
        off += g
    LANES = D // 16

    mesh = plsc.VectorSubcoreMesh(core_axis_name="c", subcore_axis_name="s")

    @functools.partial(
        pl.kernel,
        out_type=jax.ShapeDtypeStruct((N, D), jnp.float32),
        mesh=mesh,
        compiler_params=pltpu.CompilerParams(
            use_tc_tiling_on_sc=False, needs_layout_passes=False),
        scratch_types=[
            pltpu.VMEM((LP, n_chunks), jnp.int32),
            pltpu.VMEM((LP,), jnp.int32),
            pltpu.VMEM((LP,), jnp.int32),
            pltpu.VMEM((CHUNK, D), jnp.float32),
            pltpu.VMEM((CHUNK, D), jnp.float32),
            pltpu.VMEM((L, D), jnp.float32),
            pltpu.SemaphoreType.DMA,
            pltpu.SemaphoreType.DMA,
            pltpu.SemaphoreType.DMA,
            pltpu.SemaphoreType.DMA,
        ],
    )
    def emb(xt_hbm, tok_hbm, pos_hbm, out_hbm,
            idx_t, idxb0, idxb1, rows0, rows1, pos_v, g0, g1, s0, s1):
        wid = lax.axis_index("s") * nc + lax.axis_index("c")
        b0 = wid * n_chunks
        pltpu.sync_copy(xt_hbm.at[:, pl.ds(b0, n_chunks)],
                        idx_t.at[pl.ds(0, L)])
        pltpu.sync_copy(pos_hbm, pos_v)
        bufs = ((idxb0, rows0, g0, s0), (idxb1, rows1, g1, s1))
        lane = lax.iota(jnp.int32, 16)

        def build_idx(c, idxb):
            cols = jnp.full((16,), c, jnp.int32)
            for k in range(LP // 16):
                rows = lane + (16 * k)
                idxb[pl.ds(16 * k, 16)] = plsc.load_gather(idx_t, [rows, cols])

        def issue(idxb, rowsb, gsem):
            for off, gl in splits:
                pltpu.async_copy(
                    tok_hbm.at[idxb.at[pl.ds(off, gl)]],
                    rowsb.at[pl.ds(off, gl)], gsem)

        def wait_gather(idxb, rowsb, gsem):
            for off, gl in splits:
                pltpu.make_async_copy(
                    tok_hbm.at[idxb.at[pl.ds(off, gl)]],
                    rowsb.at[pl.ds(off, gl)], gsem).wait()

        def add_pos(rowsb):
            def add_body(r, carry):
                for d in range(LANES):
                    sl = pl.ds(d * 16, 16)
                    rowsb[r, sl] = rowsb[r, sl] + pos_v[r, sl]
                return carry
            lax.fori_loop(0, CHUNK, add_body, 0)

        build_idx(0, idxb0)
        issue(idxb0, rows0, g0)

        def pair_body(t, carry):
            s = t * 2
            for j in range(2):
                g = s + j
                idxb, rowsb, gsem, ssem = bufs[j]
                oidx, orows, ogsem, ossem = bufs[1 - j]

                @pl.when(g + 1 < n_chunks)
                def _issue_next():
                    @pl.when(g >= 1)
                    def _drain_store():
                        pltpu.make_async_copy(
                            orows, out_hbm.at[pl.ds(0, CHUNK)], ossem).wait()
                    build_idx(g + 1, oidx)
                    issue(oidx, orows, ogsem)

                wait_gather(idxb, rowsb, gsem)
                add_pos(rowsb)
                pltpu.async_copy(
                    rowsb,
                    out_hbm.at[pl.ds((b0 + g) * CHUNK, CHUNK)], ssem)
            return carry

        lax.fori_loop(0, n_chunks // 2, pair_body, 0)
        pltpu.make_async_copy(rows0, out_hbm.at[pl.ds(0, CHUNK)], s0).wait()
        pltpu.make_async_copy(rows1, out_hbm.at[pl.ds(0, CHUNK)], s1).wait()

    return emb


def kernel(x, token_table, pos_table):
    B, L = x.shape
    V, D = token_table.shape
    N = B * L
    xt = x.T
    emb = _build_lookup(N, V, D, L, B)
    out = emb(xt, token_table, pos_table)
    return out.reshape(B, L, D)
